# baseline (speedup 1.0000x reference)
import functools

import jax
import jax.numpy as jnp
from jax import lax
from jax.experimental import pallas as pl
from jax.experimental.pallas import tpu as pltpu
from jax.experimental.pallas import tpu_sc as plsc

_OUT_D = 128
_SEG = 42
_B = 16384
_NC, _NS = 2, 16
_NW = _NC * _NS
_BPW = _B // _NW
_CHUNK = 128
_NCHUNK = _BPW // _CHUNK
_SW = 48
_G = 16

_mesh = plsc.VectorSubcoreMesh(core_axis_name="c", subcore_axis_name="s")


@functools.partial(
    pl.kernel,
    mesh=_mesh,
    compiler_params=pltpu.CompilerParams(use_tc_tiling_on_sc=False),
    out_type=jax.ShapeDtypeStruct((_B, _OUT_D), jnp.float32),
    scratch_types=[
        pltpu.VMEM((_NCHUNK, _CHUNK), jnp.int32),
        pltpu.VMEM((_NCHUNK, _CHUNK), jnp.int32),
        pltpu.VMEM((_NCHUNK, _CHUNK), jnp.int32),
        pltpu.VMEM((64, _SW), jnp.float32),
        pltpu.VMEM((32, _OUT_D), jnp.float32),
        pltpu.VMEM((16, _SW), jnp.float32),
        pltpu.VMEM((_BPW, _OUT_D), jnp.float32),
        pltpu.SemaphoreType.DMA,
        pltpu.SemaphoreType.DMA,
    ],
)
def _recombine(v_idx_hbm, d_idx_hbm, j_idx_hbm,
               v_bank_hbm, d_bank_hbm, j_bank_hbm,
               out_hbm, vi, di, ji, vb, db, jb, rows, sem_i, sem_o):
    wid = lax.axis_index("s") * _NC + lax.axis_index("c")
    base = wid * _BPW

    cpi = [
        pltpu.async_copy(v_idx_hbm.at[pl.ds(wid * _NCHUNK, _NCHUNK)], vi, sem_i),
        pltpu.async_copy(d_idx_hbm.at[pl.ds(wid * _NCHUNK, _NCHUNK)], di, sem_i),
        pltpu.async_copy(j_idx_hbm.at[pl.ds(wid * _NCHUNK, _NCHUNK)], ji, sem_i),
        pltpu.async_copy(v_bank_hbm, vb, sem_i),
        pltpu.async_copy(d_bank_hbm, db, sem_i),
        pltpu.async_copy(j_bank_hbm, jb, sem_i),
    ]
    for cp in cpi:
        cp.wait()

    out_cps = []
    for c in range(_NCHUNK):
        def grp_body(g, carry=None, c=c):
            o16 = g * _G
            vvec = vi[c, pl.ds(o16, _G)]
            dvec = di[c, pl.ds(o16, _G)]
            jvec = ji[c, pl.ds(o16, _G)]
            for l in range(_G):
                rr = c * _CHUNK + o16 + l
                r_v = vvec[l]
                r_d = dvec[l]
                r_j = jvec[l]
                t0 = vb[r_v, pl.ds(0, 16)]
                t1 = vb[r_v, pl.ds(16, 16)]
                t2 = vb[r_v, pl.ds(32, 16)] + db[r_d, pl.ds(32, 16)]
                t3 = db[r_d, pl.ds(48, 16)]
                t4 = db[r_d, pl.ds(64, 16)]
                t5 = db[r_d, pl.ds(80, 16)] + jb[r_j, pl.ds(0, 16)]
                t6 = jb[r_j, pl.ds(16, 16)]
                t7 = jb[r_j, pl.ds(32, 16)]
                rows[rr, pl.ds(0, 16)] = t0
                rows[rr, pl.ds(16, 16)] = t1
                rows[rr, pl.ds(32, 16)] = t2
                rows[rr, pl.ds(48, 16)] = t3
                rows[rr, pl.ds(64, 16)] = t4
                rows[rr, pl.ds(80, 16)] = t5
                rows[rr, pl.ds(96, 16)] = t6
                rows[rr, pl.ds(112, 16)] = t7
            return carry

        lax.fori_loop(0, _CHUNK // _G, grp_body, 0)
        out_cps.append(pltpu.async_copy(
            rows.at[pl.ds(c * _CHUNK, _CHUNK)],
            out_hbm.at[pl.ds(base + c * _CHUNK, _CHUNK)], sem_o))
    for cp in out_cps:
        cp.wait()


def kernel(v_idx, d_idx, j_idx, v_bank, d_bank, j_bank):
    vi = v_idx.astype(jnp.int32).reshape(_NW * _NCHUNK, _CHUNK)
    di = d_idx.astype(jnp.int32).reshape(_NW * _NCHUNK, _CHUNK)
    ji = j_idx.astype(jnp.int32).reshape(_NW * _NCHUNK, _CHUNK)
    v_p = jnp.pad(v_bank, ((0, 0), (0, _SW - _SEG)))
    d_p = jnp.pad(d_bank, ((0, 0), (_SEG, _OUT_D - 2 * _SEG)))
    j_p = jnp.pad(j_bank, ((0, 0), (_SW - (_OUT_D - 2 * _SEG), 0)))
    return _recombine(vi, di, ji, v_p, d_p, j_p)


# speedup vs baseline: 1.0017x; 1.0017x over previous
import functools

import jax
import jax.numpy as jnp
from jax import lax
from jax.experimental import pallas as pl
from jax.experimental.pallas import tpu as pltpu
from jax.experimental.pallas import tpu_sc as plsc

_OUT_D = 128
_SEG = 42
_B = 16384
_NC, _NS = 2, 16
_NW = _NC * _NS
_BPW = _B // _NW
_CHUNK = 256
_NCHUNK = _BPW // _CHUNK
_SW = 48
_G = 16

_mesh = plsc.VectorSubcoreMesh(core_axis_name="c", subcore_axis_name="s")


@functools.partial(
    pl.kernel,
    mesh=_mesh,
    compiler_params=pltpu.CompilerParams(use_tc_tiling_on_sc=False, disable_bounds_checks=True, disable_semaphore_checks=True),
    out_type=jax.ShapeDtypeStruct((_B, _OUT_D), jnp.float32),
    scratch_types=[
        pltpu.VMEM((_NCHUNK, _CHUNK), jnp.int32),
        pltpu.VMEM((_NCHUNK, _CHUNK), jnp.int32),
        pltpu.VMEM((_NCHUNK, _CHUNK), jnp.int32),
        pltpu.VMEM((64, _SW), jnp.float32),
        pltpu.VMEM((32, _OUT_D), jnp.float32),
        pltpu.VMEM((16, _SW), jnp.float32),
        pltpu.VMEM((_BPW, _OUT_D), jnp.float32),
        pltpu.SemaphoreType.DMA,
        pltpu.SemaphoreType.DMA,
    ],
)
def _recombine(v_idx_hbm, d_idx_hbm, j_idx_hbm,
               v_bank_hbm, d_bank_hbm, j_bank_hbm,
               out_hbm, vi, di, ji, vb, db, jb, rows, sem_i, sem_o):
    wid = lax.axis_index("s") * _NC + lax.axis_index("c")
    base = wid * _BPW

    cpi = [
        pltpu.async_copy(v_idx_hbm.at[pl.ds(wid * _NCHUNK, _NCHUNK)], vi, sem_i),
        pltpu.async_copy(d_idx_hbm.at[pl.ds(wid * _NCHUNK, _NCHUNK)], di, sem_i),
        pltpu.async_copy(j_idx_hbm.at[pl.ds(wid * _NCHUNK, _NCHUNK)], ji, sem_i),
        pltpu.async_copy(v_bank_hbm, vb, sem_i),
        pltpu.async_copy(d_bank_hbm, db, sem_i),
        pltpu.async_copy(j_bank_hbm, jb, sem_i),
    ]
    for cp in cpi:
        cp.wait()

    out_cps = []
    for c in range(_NCHUNK):
        def grp_body(g, carry=None, c=c):
            o16 = g * _G
            vvec = vi[c, pl.ds(o16, _G)]
            dvec = di[c, pl.ds(o16, _G)]
            jvec = ji[c, pl.ds(o16, _G)]
            for l in range(_G):
                rr = c * _CHUNK + o16 + l
                r_v = vvec[l]
                r_d = dvec[l]
                r_j = jvec[l]
                t0 = vb[r_v, pl.ds(0, 16)]
                t1 = vb[r_v, pl.ds(16, 16)]
                t2 = vb[r_v, pl.ds(32, 16)] + db[r_d, pl.ds(32, 16)]
                t3 = db[r_d, pl.ds(48, 16)]
                t4 = db[r_d, pl.ds(64, 16)]
                t5 = db[r_d, pl.ds(80, 16)] + jb[r_j, pl.ds(0, 16)]
                t6 = jb[r_j, pl.ds(16, 16)]
                t7 = jb[r_j, pl.ds(32, 16)]
                rows[rr, pl.ds(0, 16)] = t0
                rows[rr, pl.ds(16, 16)] = t1
                rows[rr, pl.ds(32, 16)] = t2
                rows[rr, pl.ds(48, 16)] = t3
                rows[rr, pl.ds(64, 16)] = t4
                rows[rr, pl.ds(80, 16)] = t5
                rows[rr, pl.ds(96, 16)] = t6
                rows[rr, pl.ds(112, 16)] = t7
            return carry

        lax.fori_loop(0, _CHUNK // _G, grp_body, 0)
        out_cps.append(pltpu.async_copy(
            rows.at[pl.ds(c * _CHUNK, _CHUNK)],
            out_hbm.at[pl.ds(base + c * _CHUNK, _CHUNK)], sem_o))
    for cp in out_cps:
        cp.wait()


def kernel(v_idx, d_idx, j_idx, v_bank, d_bank, j_bank):
    vi = v_idx.astype(jnp.int32).reshape(_NW * _NCHUNK, _CHUNK)
    di = d_idx.astype(jnp.int32).reshape(_NW * _NCHUNK, _CHUNK)
    ji = j_idx.astype(jnp.int32).reshape(_NW * _NCHUNK, _CHUNK)
    v_p = jnp.pad(v_bank, ((0, 0), (0, _SW - _SEG)))
    d_p = jnp.pad(d_bank, ((0, 0), (_SEG, _OUT_D - 2 * _SEG)))
    j_p = jnp.pad(j_bank, ((0, 0), (_SW - (_OUT_D - 2 * _SEG), 0)))
    return _recombine(vi, di, ji, v_p, d_p, j_p)


# speedup vs baseline: 1.0038x; 1.0020x over previous
import functools

import jax
import jax.numpy as jnp
from jax import lax
from jax.experimental import pallas as pl
from jax.experimental.pallas import tpu as pltpu
from jax.experimental.pallas import tpu_sc as plsc

_OUT_D = 128
_SEG = 42
_B = 16384
_NC, _NS = 2, 16
_NW = _NC * _NS
_BPW = _B // _NW
_CHUNK = 256
_NCHUNK = _BPW // _CHUNK
_SW = 48
_G = 16

_mesh = plsc.VectorSubcoreMesh(core_axis_name="c", subcore_axis_name="s")


@functools.partial(
    pl.kernel,
    mesh=_mesh,
    compiler_params=pltpu.CompilerParams(use_tc_tiling_on_sc=False),
    out_type=jax.ShapeDtypeStruct((_B, _OUT_D), jnp.float32),
    scratch_types=[
        pltpu.VMEM((_NCHUNK, _CHUNK), jnp.int32),
        pltpu.VMEM((_NCHUNK, _CHUNK), jnp.int32),
        pltpu.VMEM((_NCHUNK, _CHUNK), jnp.int32),
        pltpu.VMEM((64, _SW), jnp.float32),
        pltpu.VMEM((32, _OUT_D), jnp.float32),
        pltpu.VMEM((16, _SW), jnp.float32),
        pltpu.VMEM((_BPW, _OUT_D), jnp.float32),
        pltpu.SemaphoreType.DMA,
        pltpu.SemaphoreType.DMA,
    ],
)
def _recombine(v_idx_hbm, d_idx_hbm, j_idx_hbm,
               v_bank_hbm, d_bank_hbm, j_bank_hbm,
               out_hbm, vi, di, ji, vb, db, jb, rows, sem_i, sem_o):
    wid = lax.axis_index("s") * _NC + lax.axis_index("c")
    base = wid * _BPW

    cpi = [
        pltpu.async_copy(v_idx_hbm.at[pl.ds(wid * _NCHUNK, _NCHUNK)], vi, sem_i),
        pltpu.async_copy(d_idx_hbm.at[pl.ds(wid * _NCHUNK, _NCHUNK)], di, sem_i),
        pltpu.async_copy(j_idx_hbm.at[pl.ds(wid * _NCHUNK, _NCHUNK)], ji, sem_i),
        pltpu.async_copy(v_bank_hbm, vb, sem_i),
        pltpu.async_copy(d_bank_hbm, db, sem_i),
        pltpu.async_copy(j_bank_hbm, jb, sem_i),
    ]
    for cp in cpi:
        cp.wait()

    out_cps = []
    for c in range(_NCHUNK):
        def grp_body(g, carry=None, c=c):
            o16 = g * _G
            vvec = vi[c, pl.ds(o16, _G)]
            dvec = di[c, pl.ds(o16, _G)]
            jvec = ji[c, pl.ds(o16, _G)]
            for l in range(_G):
                rr = c * _CHUNK + o16 + l
                r_v = vvec[l]
                r_d = dvec[l]
                r_j = jvec[l]
                t0 = vb[r_v, pl.ds(0, 16)]
                t1 = vb[r_v, pl.ds(16, 16)]
                t2 = vb[r_v, pl.ds(32, 16)] + db[r_d, pl.ds(32, 16)]
                t3 = db[r_d, pl.ds(48, 16)]
                t4 = db[r_d, pl.ds(64, 16)]
                t5 = db[r_d, pl.ds(80, 16)] + jb[r_j, pl.ds(0, 16)]
                t6 = jb[r_j, pl.ds(16, 16)]
                t7 = jb[r_j, pl.ds(32, 16)]
                rows[rr, pl.ds(0, 16)] = t0
                rows[rr, pl.ds(16, 16)] = t1
                rows[rr, pl.ds(32, 16)] = t2
                rows[rr, pl.ds(48, 16)] = t3
                rows[rr, pl.ds(64, 16)] = t4
                rows[rr, pl.ds(80, 16)] = t5
                rows[rr, pl.ds(96, 16)] = t6
                rows[rr, pl.ds(112, 16)] = t7
            return carry

        lax.fori_loop(0, _CHUNK // _G, grp_body, 0)
        out_cps.append(pltpu.async_copy(
            rows.at[pl.ds(c * _CHUNK, _CHUNK)],
            out_hbm.at[pl.ds(base + c * _CHUNK, _CHUNK)], sem_o))
    for cp in out_cps:
        cp.wait()


def kernel(v_idx, d_idx, j_idx, v_bank, d_bank, j_bank):
    vi = v_idx.astype(jnp.int32).reshape(_NW * _NCHUNK, _CHUNK)
    di = d_idx.astype(jnp.int32).reshape(_NW * _NCHUNK, _CHUNK)
    ji = j_idx.astype(jnp.int32).reshape(_NW * _NCHUNK, _CHUNK)
    v_p = jnp.pad(v_bank, ((0, 0), (0, _SW - _SEG)))
    d_p = jnp.pad(d_bank, ((0, 0), (_SEG, _OUT_D - 2 * _SEG)))
    j_p = jnp.pad(j_bank, ((0, 0), (_SW - (_OUT_D - 2 * _SEG), 0)))
    return _recombine(vi, di, ji, v_p, d_p, j_p)
